# baseline (speedup 1.0000x reference)
import functools

import jax
import jax.numpy as jnp
from jax import lax
from jax.experimental import pallas as pl
from jax.experimental.pallas import tpu as pltpu
from jax.experimental.pallas import tpu_sc as plsc

NUM_LAYERS = 3
D = 128
N = 10000
E = 320000

NC = 2
NS = 16
HALF = N // NC
CH = 64
NCH = 320
NCHP = 336
GB = 4
NGB = NCH // 8
ACCR = 5056
HL1 = 520
HL2 = 112
HST = 632
EXR = 632


def _sc_body(h_hbm, z_hbm, src_hbm, dst_hbm, out_hbm,
             slotSA, slotSB, slotDA, slotDB, buf, h_sh, acc_sh,
             semS, semT):
    c = lax.axis_index("c")
    s = lax.axis_index("s")

    st1 = s * HST
    pltpu.sync_copy(h_hbm.at[pl.ds(st1, HL1)], h_sh.at[pl.ds(st1, HL1)])
    st2 = jnp.minimum(st1 + HL1, N - HL2)
    pltpu.sync_copy(h_hbm.at[pl.ds(st2, HL2)], h_sh.at[pl.ds(st2, HL2)])

    @pl.when(s >= 8)
    def _():
        pltpu.sync_copy(z_hbm, acc_sh.at[pl.ds((s - 8) * EXR, EXR)])

    pltpu.sync_copy(src_hbm.at[s, pl.ds(0, GB)], slotSA)
    pltpu.sync_copy(dst_hbm.at[c, s, pl.ds(0, GB)], slotDA)
    pltpu.async_copy(src_hbm.at[s, pl.ds(GB, GB)], slotSB, semS)
    pltpu.async_copy(dst_hbm.at[c, s, pl.ds(GB, GB)], slotDB, semT)
    plsc.subcore_barrier()

    def body(g, _):
        j0 = 8 * g

        pltpu.sync_copy(h_sh.at[slotSA.at[0]], buf)
        pltpu.sync_copy(buf, acc_sh.at[slotDA.at[0]], add=True)

        pltpu.sync_copy(h_sh.at[slotSA.at[1]], buf)
        pltpu.sync_copy(buf, acc_sh.at[slotDA.at[1]], add=True)

        pltpu.sync_copy(h_sh.at[slotSA.at[2]], buf)
        pltpu.sync_copy(buf, acc_sh.at[slotDA.at[2]], add=True)

        pltpu.sync_copy(h_sh.at[slotSA.at[3]], buf)
        pltpu.sync_copy(buf, acc_sh.at[slotDA.at[3]], add=True)

        pltpu.make_async_copy(src_hbm.at[s, pl.ds(GB, GB)], slotSB, semS).wait()
        pltpu.make_async_copy(dst_hbm.at[c, s, pl.ds(GB, GB)], slotDB, semT).wait()
        pltpu.async_copy(src_hbm.at[s, pl.ds(j0 + 8, GB)], slotSA, semS)
        pltpu.async_copy(dst_hbm.at[c, s, pl.ds(j0 + 8, GB)], slotDA, semT)

        pltpu.sync_copy(h_sh.at[slotSB.at[0]], buf)
        pltpu.sync_copy(buf, acc_sh.at[slotDB.at[0]], add=True)

        pltpu.sync_copy(h_sh.at[slotSB.at[1]], buf)
        pltpu.sync_copy(buf, acc_sh.at[slotDB.at[1]], add=True)

        pltpu.sync_copy(h_sh.at[slotSB.at[2]], buf)
        pltpu.sync_copy(buf, acc_sh.at[slotDB.at[2]], add=True)

        pltpu.sync_copy(h_sh.at[slotSB.at[3]], buf)
        pltpu.sync_copy(buf, acc_sh.at[slotDB.at[3]], add=True)

        pltpu.make_async_copy(src_hbm.at[s, pl.ds(0, GB)], slotSA, semS).wait()
        pltpu.make_async_copy(dst_hbm.at[c, s, pl.ds(GB, GB)], slotDA, semT).wait()
        pltpu.async_copy(src_hbm.at[s, pl.ds(j0 + 12, GB)], slotSB, semS)
        pltpu.async_copy(dst_hbm.at[c, s, pl.ds(j0 + 12, GB)], slotDB, semT)
        return 0

    lax.fori_loop(0, NGB, body, 0)

    pltpu.make_async_copy(src_hbm.at[s, pl.ds(0, GB)], slotSB, semS).wait()
    pltpu.make_async_copy(dst_hbm.at[c, s, pl.ds(GB, GB)], slotDB, semT).wait()
    plsc.subcore_barrier()

    @pl.when(s < 8)
    def _():
        pltpu.sync_copy(acc_sh.at[pl.ds(s * EXR, EXR)],
                        out_hbm.at[c, pl.ds(s * EXR, EXR)])


_sc_segment_sum = functools.partial(
    pl.kernel,
    mesh=plsc.VectorSubcoreMesh(core_axis_name="c", subcore_axis_name="s"),
    out_type=jax.ShapeDtypeStruct((NC, ACCR, D), jnp.float32),
    scratch_types=[
        pltpu.VMEM((GB, CH), jnp.int32),
        pltpu.VMEM((GB, CH), jnp.int32),
        pltpu.VMEM((GB, CH), jnp.int32),
        pltpu.VMEM((GB, CH), jnp.int32),
        pltpu.VMEM((CH, D), jnp.float32),
        pltpu.VMEM_SHARED((N, D), jnp.float32),
        pltpu.VMEM_SHARED((ACCR, D), jnp.float32),
        pltpu.SemaphoreType.DMA,
        pltpu.SemaphoreType.DMA,
    ],
)(_sc_body)


def _tc_body(h_ref, a_ref, w1_ref, b1_ref, w2_ref, b2_ref,
             g_ref, be_ref, o_ref):
    agg = jnp.concatenate([a_ref[0, :HALF], a_ref[1, :HALF]], axis=0)
    z = h_ref[...] + agg
    z = jnp.dot(z, w1_ref[...], preferred_element_type=jnp.float32) + b1_ref[...]
    z = jnp.where(z > 0, z, 0.01 * z)
    z = jnp.dot(z, w2_ref[...], preferred_element_type=jnp.float32) + b2_ref[...]
    z = jnp.maximum(z, 0.0)
    mean = jnp.sum(z, axis=0, keepdims=True) * (1.0 / N)
    var = jnp.sum(z * z, axis=0, keepdims=True) * (1.0 / N) - mean * mean
    o_ref[...] = (z - mean) * lax.rsqrt(var + 1e-4) * g_ref[...] + be_ref[...]


def _tc_layer(h, agg, W1, b1, W2, b2, gamma, beta):
    return pl.pallas_call(
        _tc_body,
        out_shape=jax.ShapeDtypeStruct((N, D), jnp.float32),
    )(h, agg, W1, b1.reshape(1, D), W2, b2.reshape(1, D),
      gamma.reshape(1, D), beta.reshape(1, D))


def kernel(x, edge_index, W1, b1, W2, b2, gamma, beta):
    pad = NS * NCH * CH - E
    src = jnp.concatenate(
        [edge_index[0].astype(jnp.int32),
         jnp.zeros((pad,), jnp.int32)]).reshape(NS, NCH, CH)
    src = jnp.concatenate(
        [src, jnp.zeros((NS, NCHP - NCH, CH), jnp.int32)], axis=1)
    dstp = jnp.concatenate(
        [edge_index[1].astype(jnp.int32),
         jnp.full((pad,), N, jnp.int32)]).reshape(NS, NCH, CH)
    tid = HALF + jnp.arange(NS, dtype=jnp.int32).reshape(NS, 1, 1)
    d0 = jnp.where(dstp < HALF, dstp, tid)
    d1 = jnp.where((dstp >= HALF) & (dstp < N), dstp - HALF, tid)
    dst = jnp.stack([d0, d1])
    dst = jnp.concatenate(
        [dst, jnp.full((NC, NS, NCHP - NCH, CH), HALF, jnp.int32)], axis=2)
    zrows = jnp.zeros((EXR, D), jnp.float32)
    h = x
    for i in range(NUM_LAYERS):
        agg = _sc_segment_sum(h, zrows, src, dst)
        h = _tc_layer(h, agg, W1[i], b1[i], W2[i], b2[i], gamma[i], beta[i])
    return h


# speedup vs baseline: 1.4339x; 1.4339x over previous
import functools

import jax
import jax.numpy as jnp
from jax import lax
from jax.experimental import pallas as pl
from jax.experimental.pallas import tpu as pltpu
from jax.experimental.pallas import tpu_sc as plsc

NUM_LAYERS = 3
D = 128
N = 10000
E = 320000

NC = 2
NS = 16
HALF = N // NC
CH = 32
NCH = 640
NCHP = 656
GB = 4
NGB = NCH // 8
ACCR = 5056
HL1 = 520
HL2 = 112
HST = 632
EXR = 632


def _sc_body(h_hbm, z_hbm, src_hbm, dst_hbm, out_hbm,
             slotSA, slotSB, slotDA, slotDB, buf0, buf1, h_sh, acc_sh,
             semA, semB, semS, semT):
    c = lax.axis_index("c")
    s = lax.axis_index("s")

    st1 = s * HST
    pltpu.sync_copy(h_hbm.at[pl.ds(st1, HL1)], h_sh.at[pl.ds(st1, HL1)])
    st2 = jnp.minimum(st1 + HL1, N - HL2)
    pltpu.sync_copy(h_hbm.at[pl.ds(st2, HL2)], h_sh.at[pl.ds(st2, HL2)])

    @pl.when(s >= 8)
    def _():
        pltpu.sync_copy(z_hbm, acc_sh.at[pl.ds((s - 8) * EXR, EXR)])

    pltpu.sync_copy(src_hbm.at[s, pl.ds(0, GB)], slotSA)
    pltpu.sync_copy(dst_hbm.at[c, s, pl.ds(0, GB)], slotDA)
    pltpu.async_copy(src_hbm.at[s, pl.ds(GB, GB)], slotSB, semS)
    pltpu.async_copy(dst_hbm.at[c, s, pl.ds(GB, GB)], slotDB, semT)
    plsc.subcore_barrier()
    pltpu.async_copy(h_sh.at[slotSA.at[0]], buf0, semA)

    def body(g, _):
        j0 = 8 * g

        pltpu.async_copy(h_sh.at[slotSA.at[1]], buf1, semB)
        pltpu.make_async_copy(h_sh.at[slotSA.at[0]], buf0, semA).wait()
        pltpu.sync_copy(buf0, acc_sh.at[slotDA.at[0]], add=True)

        pltpu.async_copy(h_sh.at[slotSA.at[2]], buf0, semA)
        pltpu.make_async_copy(h_sh.at[slotSA.at[1]], buf1, semB).wait()
        pltpu.sync_copy(buf1, acc_sh.at[slotDA.at[1]], add=True)

        pltpu.async_copy(h_sh.at[slotSA.at[3]], buf1, semB)
        pltpu.make_async_copy(h_sh.at[slotSA.at[2]], buf0, semA).wait()
        pltpu.sync_copy(buf0, acc_sh.at[slotDA.at[2]], add=True)

        pltpu.make_async_copy(src_hbm.at[s, pl.ds(GB, GB)], slotSB, semS).wait()
        pltpu.async_copy(h_sh.at[slotSB.at[0]], buf0, semA)
        pltpu.make_async_copy(h_sh.at[slotSA.at[3]], buf1, semB).wait()
        pltpu.async_copy(src_hbm.at[s, pl.ds(j0 + 8, GB)], slotSA, semS)
        pltpu.make_async_copy(dst_hbm.at[c, s, pl.ds(GB, GB)], slotDB, semT).wait()
        pltpu.sync_copy(buf1, acc_sh.at[slotDA.at[3]], add=True)
        pltpu.async_copy(dst_hbm.at[c, s, pl.ds(j0 + 8, GB)], slotDA, semT)

        pltpu.async_copy(h_sh.at[slotSB.at[1]], buf1, semB)
        pltpu.make_async_copy(h_sh.at[slotSB.at[0]], buf0, semA).wait()
        pltpu.sync_copy(buf0, acc_sh.at[slotDB.at[0]], add=True)

        pltpu.async_copy(h_sh.at[slotSB.at[2]], buf0, semA)
        pltpu.make_async_copy(h_sh.at[slotSB.at[1]], buf1, semB).wait()
        pltpu.sync_copy(buf1, acc_sh.at[slotDB.at[1]], add=True)

        pltpu.async_copy(h_sh.at[slotSB.at[3]], buf1, semB)
        pltpu.make_async_copy(h_sh.at[slotSB.at[2]], buf0, semA).wait()
        pltpu.sync_copy(buf0, acc_sh.at[slotDB.at[2]], add=True)

        pltpu.make_async_copy(src_hbm.at[s, pl.ds(0, GB)], slotSA, semS).wait()
        pltpu.async_copy(h_sh.at[slotSA.at[0]], buf0, semA)
        pltpu.make_async_copy(h_sh.at[slotSB.at[3]], buf1, semB).wait()
        pltpu.async_copy(src_hbm.at[s, pl.ds(j0 + 12, GB)], slotSB, semS)
        pltpu.make_async_copy(dst_hbm.at[c, s, pl.ds(GB, GB)], slotDA, semT).wait()
        pltpu.sync_copy(buf1, acc_sh.at[slotDB.at[3]], add=True)
        pltpu.async_copy(dst_hbm.at[c, s, pl.ds(j0 + 12, GB)], slotDB, semT)
        return 0

    lax.fori_loop(0, NGB, body, 0)

    pltpu.make_async_copy(src_hbm.at[s, pl.ds(0, GB)], slotSB, semS).wait()
    pltpu.make_async_copy(dst_hbm.at[c, s, pl.ds(GB, GB)], slotDB, semT).wait()
    pltpu.make_async_copy(h_sh.at[slotSA.at[0]], buf0, semA).wait()
    plsc.subcore_barrier()

    @pl.when(s < 8)
    def _():
        pltpu.sync_copy(acc_sh.at[pl.ds(s * EXR, EXR)],
                        out_hbm.at[c, pl.ds(s * EXR, EXR)])


_sc_segment_sum = functools.partial(
    pl.kernel,
    mesh=plsc.VectorSubcoreMesh(core_axis_name="c", subcore_axis_name="s"),
    out_type=jax.ShapeDtypeStruct((NC, ACCR, D), jnp.float32),
    scratch_types=[
        pltpu.VMEM((GB, CH), jnp.int32),
        pltpu.VMEM((GB, CH), jnp.int32),
        pltpu.VMEM((GB, CH), jnp.int32),
        pltpu.VMEM((GB, CH), jnp.int32),
        pltpu.VMEM((CH, D), jnp.float32),
        pltpu.VMEM((CH, D), jnp.float32),
        pltpu.VMEM_SHARED((N, D), jnp.float32),
        pltpu.VMEM_SHARED((ACCR, D), jnp.float32),
        pltpu.SemaphoreType.DMA,
        pltpu.SemaphoreType.DMA,
        pltpu.SemaphoreType.DMA,
        pltpu.SemaphoreType.DMA,
    ],
)(_sc_body)


def _tc_body(h_ref, a_ref, w1_ref, b1_ref, w2_ref, b2_ref,
             g_ref, be_ref, o_ref):
    agg = jnp.concatenate([a_ref[0, :HALF], a_ref[1, :HALF]], axis=0)
    z = h_ref[...] + agg
    z = jnp.dot(z, w1_ref[...], preferred_element_type=jnp.float32) + b1_ref[...]
    z = jnp.where(z > 0, z, 0.01 * z)
    z = jnp.dot(z, w2_ref[...], preferred_element_type=jnp.float32) + b2_ref[...]
    z = jnp.maximum(z, 0.0)
    mean = jnp.sum(z, axis=0, keepdims=True) * (1.0 / N)
    var = jnp.sum(z * z, axis=0, keepdims=True) * (1.0 / N) - mean * mean
    o_ref[...] = (z - mean) * lax.rsqrt(var + 1e-4) * g_ref[...] + be_ref[...]


def _tc_layer(h, agg, W1, b1, W2, b2, gamma, beta):
    return pl.pallas_call(
        _tc_body,
        out_shape=jax.ShapeDtypeStruct((N, D), jnp.float32),
    )(h, agg, W1, b1.reshape(1, D), W2, b2.reshape(1, D),
      gamma.reshape(1, D), beta.reshape(1, D))


def kernel(x, edge_index, W1, b1, W2, b2, gamma, beta):
    pad = NS * NCH * CH - E
    src = jnp.concatenate(
        [edge_index[0].astype(jnp.int32),
         jnp.zeros((pad,), jnp.int32)]).reshape(NS, NCH, CH)
    src = jnp.concatenate(
        [src, jnp.zeros((NS, NCHP - NCH, CH), jnp.int32)], axis=1)
    dstp = jnp.concatenate(
        [edge_index[1].astype(jnp.int32),
         jnp.full((pad,), N, jnp.int32)]).reshape(NS, NCH, CH)
    tid = HALF + jnp.arange(NS, dtype=jnp.int32).reshape(NS, 1, 1)
    d0 = jnp.where(dstp < HALF, dstp, tid)
    d1 = jnp.where((dstp >= HALF) & (dstp < N), dstp - HALF, tid)
    dst = jnp.stack([d0, d1])
    dst = jnp.concatenate(
        [dst, jnp.full((NC, NS, NCHP - NCH, CH), HALF, jnp.int32)], axis=2)
    zrows = jnp.zeros((EXR, D), jnp.float32)
    h = x
    for i in range(NUM_LAYERS):
        agg = _sc_segment_sum(h, zrows, src, dst)
        h = _tc_layer(h, agg, W1[i], b1[i], W2[i], b2[i], gamma[i], beta[i])
    return h
